# Initial kernel scaffold; baseline (speedup 1.0000x reference)
import jax
import jax.numpy as jnp
from jax.experimental import pallas as pl


def kernel(expert_embeddings, mask, W1, b1, W2, b2, Wc1, bc1, Wc2, bc2):
    raise NotImplementedError("write your pallas kernel here")


# speedup vs baseline: 3.0451x; 3.0451x over previous
import functools

import jax
import jax.numpy as jnp
from jax.experimental import pallas as pl


def _gating_block_kernel(emb_ref, maskf_ref, w1_ref, b1_ref, w2_ref,
                         b2_ref, wc1_ref, bc1_ref, wc2_ref, bc2_ref,
                         prob_ref, weights_ref, *, n_experts):
    w1 = w1_ref[:]
    b1 = b1_ref[:]
    w2 = w2_ref[:]
    maskf = maskf_ref[:]

    logits_cols = []
    for e in range(n_experts):
        x_e = emb_ref[e]
        h = jnp.maximum(jnp.dot(x_e, w1) + b1, 0.0)
        logits_cols.append(jnp.sum(h * w2, axis=1))
    logits = jnp.stack(logits_cols, axis=1) + b2_ref[0, 0]

    neg = jnp.float32(-1e30)
    ml = jnp.where(maskf > 0.0, logits, neg)
    m = jnp.max(ml, axis=1, keepdims=True)
    ex = jnp.exp(ml - m) * maskf
    denom = jnp.sum(ex, axis=1, keepdims=True)
    any_valid = denom > 0.0
    weights = jnp.where(any_valid, ex / jnp.where(any_valid, denom, 1.0), 0.0)
    weights_ref[:] = weights

    fused = weights[:, 0:1] * emb_ref[0]
    for e in range(1, n_experts):
        fused = fused + weights[:, e:e + 1] * emb_ref[e]

    hc = jnp.maximum(jnp.dot(fused, wc1_ref[:]) + bc1_ref[:], 0.0)
    z = jnp.sum(hc * wc2_ref[:], axis=1, keepdims=True) + bc2_ref[0, 0]
    prob_ref[:] = jax.nn.sigmoid(z)


@functools.partial(jax.jit, static_argnames=())
def kernel(expert_embeddings, mask, W1, b1, W2, b2, Wc1, bc1, Wc2, bc2):
    E, B, D = expert_embeddings.shape
    H = W1.shape[1]
    bm = min(256, B)
    assert B % bm == 0
    grid = (B // bm,)

    maskf = mask.astype(jnp.float32)
    b1r = b1.reshape(1, H)
    w2r = W2.reshape(1, H)
    bc1r = bc1.reshape(1, H)
    wc2r = Wc2.reshape(1, H)
    b2r = b2.reshape(1, 1)
    bc2r = bc2.reshape(1, 1)

    out_shapes = (
        jax.ShapeDtypeStruct((B, 1), jnp.float32),
        jax.ShapeDtypeStruct((B, E), jnp.float32),
    )
    in_specs = [
        pl.BlockSpec((E, bm, D), lambda i: (0, i, 0)),
        pl.BlockSpec((bm, E), lambda i: (i, 0)),
        pl.BlockSpec((D, H), lambda i: (0, 0)),
        pl.BlockSpec((1, H), lambda i: (0, 0)),
        pl.BlockSpec((1, H), lambda i: (0, 0)),
        pl.BlockSpec((1, 1), lambda i: (0, 0)),
        pl.BlockSpec((D, H), lambda i: (0, 0)),
        pl.BlockSpec((1, H), lambda i: (0, 0)),
        pl.BlockSpec((1, H), lambda i: (0, 0)),
        pl.BlockSpec((1, 1), lambda i: (0, 0)),
    ]
    out_specs = (
        pl.BlockSpec((bm, 1), lambda i: (i, 0)),
        pl.BlockSpec((bm, E), lambda i: (i, 0)),
    )

    final_prob, weights = pl.pallas_call(
        functools.partial(_gating_block_kernel, n_experts=E),
        grid=grid,
        in_specs=in_specs,
        out_specs=out_specs,
        out_shape=out_shapes,
    )(expert_embeddings, maskf, W1, b1r, w2r, b2r, Wc1, bc1r, wc2r, bc2r)

    return final_prob, weights
